# baseline (speedup 1.0000x reference)
import functools

import jax
import jax.numpy as jnp
from jax import lax
from jax.experimental import pallas as pl
from jax.experimental.pallas import tpu as pltpu
from jax.experimental.pallas import tpu_sc as plsc

NC = 2
NS = 16
NW = NC * NS
L = 16

CHUNK = 64
NBUF = 4
NIB = 8


CH_H = 128


def _hist_body(nh, rem_h, dpad, e3_hbm, degp_hbm, idx_v, idt_v, ones_v, zb_v,
               acc_sh):
    c = lax.axis_index("c")
    s = lax.axis_index("s")
    gid = c * NS + s
    seg = dpad // NS
    for i in range(CH_H // L):
        ones_v[pl.ds(i * L, L)] = jnp.ones((L,), jnp.float32)
    for i in range(seg // L):
        zb_v[pl.ds(i * L, L)] = jnp.zeros((L,), jnp.float32)
    pltpu.sync_copy(zb_v, acc_sh.at[pl.ds(s * seg, seg)])
    pltpu.sync_copy(e3_hbm.at[1, pl.ds(gid * nh, nh)], idx_v)

    @pl.when(gid < rem_h)
    def _():
        pltpu.sync_copy(e3_hbm.at[1, NW * nh + gid], idt_v)

    plsc.subcore_barrier()

    def step(j, carry):
        pltpu.sync_copy(ones_v, acc_sh.at[idx_v.at[j, 0]], add=True)
        return carry

    lax.fori_loop(0, nh, step, 0)

    @pl.when(gid < rem_h)
    def _():
        pltpu.sync_copy(ones_v, acc_sh.at[idt_v.at[0]], add=True)

    plsc.subcore_barrier()
    pltpu.sync_copy(acc_sh.at[pl.ds(s * seg, seg)],
                    degp_hbm.at[c, 0, pl.ds(s * seg, seg)])


def kernel(V, E, X, W, b):
    n, d = X.shape
    e_n = E.shape[1]

    assert e_n % CHUNK == 0 and e_n % CH_H == 0
    ncht = e_n // CHUNK
    nch = ncht // NW
    rem = ncht - nch * NW
    main = (nch // NIB) * NIB
    ntail = nch - main
    dpad = -(-n // (NS * CH_H)) * (NS * CH_H)
    apad = -(-n // (NS * 8)) * (NS * 8)

    e4 = E.reshape(2, ncht, 1, CHUNK)
    ncht_h = e_n // CH_H
    nh = ncht_h // NW
    rem_h = ncht_h - nh * NW
    e3 = E.reshape(2, ncht_h, 1, CH_H)

    mesh = plsc.VectorSubcoreMesh(core_axis_name="c", subcore_axis_name="s")
    hist = pl.kernel(
        functools.partial(_hist_body, nh, rem_h, dpad),
        out_type=jax.ShapeDtypeStruct((NC, 1, dpad), jnp.float32),
        mesh=mesh,
        scratch_types=[
            pltpu.VMEM((nh, 1, CH_H), jnp.int32),
            pltpu.VMEM((1, CH_H), jnp.int32),
            pltpu.VMEM((CH_H,), jnp.float32),
            pltpu.VMEM((dpad // NS,), jnp.float32),
            pltpu.VMEM_SHARED((dpad,), jnp.float32),
        ],
    )
    degp = hist(e3)

    def _inv_body(degp_ref, inv_ref):
        dsum = degp_ref[0, 0:1, :] + degp_ref[1, 0:1, :]
        inv_ref[...] = lax.rsqrt(jnp.maximum(dsum, 1.0)).reshape(dpad, 1)

    inv_colp = pl.pallas_call(
        _inv_body,
        out_shape=jax.ShapeDtypeStruct((dpad, 1), jnp.float32),
    )(degp)
    inv_col = inv_colp[:n]

    rb = 1000
    grid = n // rb

    def _mm_body(x_ref, w_ref, y_ref):
        y_ref[...] = jnp.dot(x_ref[...], w_ref[...],
                             preferred_element_type=jnp.float32)

    y = pl.pallas_call(
        _mm_body,
        grid=(grid,),
        in_specs=[
            pl.BlockSpec((rb, d), lambda i: (i, 0)),
            pl.BlockSpec((d, d), lambda i: (0, 0)),
        ],
        out_specs=pl.BlockSpec((rb, d), lambda i: (i, 0)),
        out_shape=jax.ShapeDtypeStruct((n, d), jnp.float32),
    )(X, W)

    def _scale_body(y_ref, inv_ref, ys_ref):
        ys_ref[...] = y_ref[...] * inv_ref[...]

    ys = pl.pallas_call(
        _scale_body,
        out_shape=jax.ShapeDtypeStruct((n, d), jnp.float32),
    )(y, inv_col)

    def _agg(ys_hbm, e4_hbm, accp_hbm, iv, rows, acc_sh, isem, gsem, ssem):
        c = lax.axis_index("c")
        s = lax.axis_index("s")
        gid = c * NS + s
        z16 = jnp.zeros((L,), jnp.float32)
        r0 = rows[0]

        def zrow(r, carry):
            for cc in range(d // L):
                r0[r, pl.ds(cc * L, L)] = z16
            return carry

        lax.fori_loop(0, CHUNK, zrow, 0)

        zseg = apad // NS
        off = 0
        while off < zseg:
            sz = min(CHUNK, zseg - off)
            pltpu.sync_copy(r0.at[pl.ds(0, sz)],
                            acc_sh.at[pl.ds(s * zseg + off, sz)])
            off += sz

        def idx_start(jc, k):
            m = gid + jc * NW
            pltpu.async_copy(e4_hbm.at[0, m], iv[k].at[pl.ds(0, 1)], isem[k])
            pltpu.async_copy(e4_hbm.at[1, m], iv[k].at[pl.ds(1, 1)], isem[k])

        def idx_wait(k):
            pltpu.make_async_copy(e4_hbm.at[0, 0], iv[k].at[pl.ds(0, 1)],
                                  isem[k]).wait()
            pltpu.make_async_copy(e4_hbm.at[0, 0], iv[k].at[pl.ds(1, 1)],
                                  isem[k]).wait()

        def g_start(k, b):
            pltpu.async_copy(ys_hbm.at[iv[k].at[0]], rows[b], gsem[b])

        def g_wait(b):
            pltpu.make_async_copy(ys_hbm.at[iv[0].at[0]], rows[b],
                                  gsem[b]).wait()

        def s_start(k, b):
            pltpu.async_copy(rows[b], acc_sh.at[iv[k].at[1]], ssem[b],
                             add=True)

        def s_wait(b):
            pltpu.make_async_copy(rows[b], acc_sh.at[iv[0].at[1]],
                                  ssem[b]).wait()

        for k in range(NBUF):
            idx_start(k, k)
        for k in range(2):
            idx_wait(k)
            g_start(k, k)
        plsc.subcore_barrier()

        def step(jj, carry):
            j = jj * NIB
            for b8 in range(NIB):
                jc = j + b8
                b = b8 % NBUF
                g_wait(b)
                s_start(b8, b)

                @pl.when(jc + NBUF < main)
                def _():
                    idx_start(jc + NBUF, (b8 + NBUF) % NIB)

                @pl.when(jc + 2 < main)
                def _():
                    @pl.when(jc >= 2)
                    def _():
                        s_wait((b8 + 2) % NBUF)
                    idx_wait((b8 + 2) % NIB)
                    g_start((b8 + 2) % NIB, (b8 + 2) % NBUF)

            return carry

        lax.fori_loop(0, main // NIB, step, 0)

        for t in range(ntail):
            idx_start(main + t, t)
        for t in range(ntail):
            idx_wait(t)
            s_wait(t % NBUF)
            g_start(t, t % NBUF)

        @pl.when(gid < rem)
        def _():
            idx_start(main + ntail, ntail)

        for t in range(ntail):
            g_wait(t % NBUF)
            s_start(t, t % NBUF)

        @pl.when(gid < rem)
        def _():
            idx_wait(ntail)
            s_wait(0)
            g_start(ntail, 0)
            g_wait(0)
            s_start(ntail, 0)

        for b in range(NBUF):
            s_wait(b)
        plsc.subcore_barrier()

        wseg = apad // NS
        pltpu.sync_copy(acc_sh.at[pl.ds(s * wseg, wseg)],
                        accp_hbm.at[c, pl.ds(s * wseg, wseg)])

    agg = pl.kernel(
        _agg,
        out_type=jax.ShapeDtypeStruct((NC, apad, d), jnp.float32),
        mesh=mesh,
        scratch_types=[
            [pltpu.VMEM((2, CHUNK), jnp.int32) for _ in range(NIB)],
            [pltpu.VMEM((CHUNK, d), jnp.float32) for _ in range(NBUF)],
            pltpu.VMEM_SHARED((apad, d), jnp.float32),
            [pltpu.SemaphoreType.DMA for _ in range(NIB)],
            [pltpu.SemaphoreType.DMA for _ in range(NBUF)],
            [pltpu.SemaphoreType.DMA for _ in range(NBUF)],
        ],
    )
    accp = agg(ys, e4)

    def _fin_body(accp_ref, inv_ref, b_ref, out_ref):
        a = accp_ref[0] + accp_ref[1]
        out_ref[...] = a[:out_ref.shape[0]] * inv_ref[...] + b_ref[...]

    out = pl.pallas_call(
        _fin_body,
        out_shape=jax.ShapeDtypeStruct((n, d), jnp.float32),
    )(accp, inv_col, b.reshape(1, d))
    return out


# speedup vs baseline: 1.0212x; 1.0212x over previous
import functools

import jax
import jax.numpy as jnp
from jax import lax
from jax.experimental import pallas as pl
from jax.experimental.pallas import tpu as pltpu
from jax.experimental.pallas import tpu_sc as plsc

NC = 2
NS = 16
NW = NC * NS
L = 16

CHUNK = 64
NBUF = 4
NIB = 8


CH_H = 128


def _hist_body(nh, rem_h, dpad, e3_hbm, degp_hbm, idx_v, idt_v, ones_v, zb_v,
               acc_sh):
    c = lax.axis_index("c")
    s = lax.axis_index("s")
    gid = c * NS + s
    seg = dpad // NS
    for i in range(CH_H // L):
        ones_v[pl.ds(i * L, L)] = jnp.ones((L,), jnp.float32)
    for i in range(seg // L):
        zb_v[pl.ds(i * L, L)] = jnp.zeros((L,), jnp.float32)
    pltpu.sync_copy(zb_v, acc_sh.at[pl.ds(s * seg, seg)])
    pltpu.sync_copy(e3_hbm.at[1, pl.ds(gid * nh, nh)], idx_v)

    @pl.when(gid < rem_h)
    def _():
        pltpu.sync_copy(e3_hbm.at[1, NW * nh + gid], idt_v)

    plsc.subcore_barrier()

    def step(j, carry):
        pltpu.sync_copy(ones_v, acc_sh.at[idx_v.at[j, 0]], add=True)
        return carry

    lax.fori_loop(0, nh, step, 0)

    @pl.when(gid < rem_h)
    def _():
        pltpu.sync_copy(ones_v, acc_sh.at[idt_v.at[0]], add=True)

    plsc.subcore_barrier()
    pltpu.sync_copy(acc_sh.at[pl.ds(s * seg, seg)],
                    degp_hbm.at[c, 0, pl.ds(s * seg, seg)])


def kernel(V, E, X, W, b):
    n, d = X.shape
    e_n = E.shape[1]

    assert e_n % CHUNK == 0 and e_n % CH_H == 0
    ncht = e_n // CHUNK
    nch = ncht // NW
    rem = ncht - nch * NW
    main = (nch // NIB) * NIB
    ntail = nch - main
    dpad = -(-n // (NS * CH_H)) * (NS * CH_H)
    apad = -(-n // (NS * 8)) * (NS * 8)

    e4 = E.reshape(2, ncht, 1, CHUNK)
    ncht_h = e_n // CH_H
    nh = ncht_h // NW
    rem_h = ncht_h - nh * NW
    e3 = E.reshape(2, ncht_h, 1, CH_H)

    mesh = plsc.VectorSubcoreMesh(core_axis_name="c", subcore_axis_name="s")
    hist = pl.kernel(
        functools.partial(_hist_body, nh, rem_h, dpad),
        out_type=jax.ShapeDtypeStruct((NC, 1, dpad), jnp.float32),
        mesh=mesh,
        scratch_types=[
            pltpu.VMEM((nh, 1, CH_H), jnp.int32),
            pltpu.VMEM((1, CH_H), jnp.int32),
            pltpu.VMEM((CH_H,), jnp.float32),
            pltpu.VMEM((dpad // NS,), jnp.float32),
            pltpu.VMEM_SHARED((dpad,), jnp.float32),
        ],
    )
    degp = hist(e3)

    def _inv_body(degp_ref, inv_ref):
        dsum = degp_ref[0, 0:1, :] + degp_ref[1, 0:1, :]
        inv_ref[...] = lax.rsqrt(jnp.maximum(dsum, 1.0))

    inv_row = pl.pallas_call(
        _inv_body,
        out_shape=jax.ShapeDtypeStruct((1, dpad), jnp.float32),
    )(degp)
    inv_col = inv_row.reshape(dpad, 1)[:n]

    rb = 1000
    grid = n // rb

    def _mm_body(x_ref, w_ref, y_ref):
        y_ref[...] = jnp.dot(x_ref[...], w_ref[...],
                             preferred_element_type=jnp.float32)

    y = pl.pallas_call(
        _mm_body,
        grid=(grid,),
        in_specs=[
            pl.BlockSpec((rb, d), lambda i: (i, 0)),
            pl.BlockSpec((d, d), lambda i: (0, 0)),
        ],
        out_specs=pl.BlockSpec((rb, d), lambda i: (i, 0)),
        out_shape=jax.ShapeDtypeStruct((n, d), jnp.float32),
    )(X, W)

    def _scale_body(y_ref, inv_ref, ys_ref):
        ys_ref[...] = y_ref[...] * inv_ref[...]

    ys = pl.pallas_call(
        _scale_body,
        out_shape=jax.ShapeDtypeStruct((n, d), jnp.float32),
    )(y, inv_col)

    def _agg(ys_hbm, e4_hbm, accp_hbm, iv, rows, acc_sh, isem, gsem, ssem):
        c = lax.axis_index("c")
        s = lax.axis_index("s")
        gid = c * NS + s
        z16 = jnp.zeros((L,), jnp.float32)
        r0 = rows[0]

        def zrow(r, carry):
            for cc in range(d // L):
                r0[r, pl.ds(cc * L, L)] = z16
            return carry

        lax.fori_loop(0, CHUNK, zrow, 0)

        zseg = apad // NS
        off = 0
        while off < zseg:
            sz = min(CHUNK, zseg - off)
            pltpu.sync_copy(r0.at[pl.ds(0, sz)],
                            acc_sh.at[pl.ds(s * zseg + off, sz)])
            off += sz

        def idx_start(jc, k):
            m = gid + jc * NW
            pltpu.async_copy(e4_hbm.at[0, m], iv[k].at[pl.ds(0, 1)], isem[k])
            pltpu.async_copy(e4_hbm.at[1, m], iv[k].at[pl.ds(1, 1)], isem[k])

        def idx_wait(k):
            pltpu.make_async_copy(e4_hbm.at[0, 0], iv[k].at[pl.ds(0, 1)],
                                  isem[k]).wait()
            pltpu.make_async_copy(e4_hbm.at[0, 0], iv[k].at[pl.ds(1, 1)],
                                  isem[k]).wait()

        def g_start(k, b):
            pltpu.async_copy(ys_hbm.at[iv[k].at[0]], rows[b], gsem[b])

        def g_wait(b):
            pltpu.make_async_copy(ys_hbm.at[iv[0].at[0]], rows[b],
                                  gsem[b]).wait()

        def s_start(k, b):
            pltpu.async_copy(rows[b], acc_sh.at[iv[k].at[1]], ssem[b],
                             add=True)

        def s_wait(b):
            pltpu.make_async_copy(rows[b], acc_sh.at[iv[0].at[1]],
                                  ssem[b]).wait()

        for k in range(NBUF):
            idx_start(k, k)
        for k in range(2):
            idx_wait(k)
            g_start(k, k)
        plsc.subcore_barrier()

        def step(jj, carry):
            j = jj * NIB
            for b8 in range(NIB):
                jc = j + b8
                b = b8 % NBUF
                g_wait(b)
                s_start(b8, b)

                @pl.when(jc + NBUF < main)
                def _():
                    idx_start(jc + NBUF, (b8 + NBUF) % NIB)

                @pl.when(jc + 2 < main)
                def _():
                    @pl.when(jc >= 2)
                    def _():
                        s_wait((b8 + 2) % NBUF)
                    idx_wait((b8 + 2) % NIB)
                    g_start((b8 + 2) % NIB, (b8 + 2) % NBUF)

            return carry

        lax.fori_loop(0, main // NIB, step, 0)

        for t in range(ntail):
            idx_start(main + t, t)
        for t in range(ntail):
            idx_wait(t)
            s_wait(t % NBUF)
            g_start(t, t % NBUF)

        @pl.when(gid < rem)
        def _():
            idx_start(main + ntail, ntail)

        for t in range(ntail):
            g_wait(t % NBUF)
            s_start(t, t % NBUF)

        @pl.when(gid < rem)
        def _():
            idx_wait(ntail)
            s_wait(0)
            g_start(ntail, 0)
            g_wait(0)
            s_start(ntail, 0)

        for b in range(NBUF):
            s_wait(b)
        plsc.subcore_barrier()

        wseg = apad // NS
        pltpu.sync_copy(acc_sh.at[pl.ds(s * wseg, wseg)],
                        accp_hbm.at[c, pl.ds(s * wseg, wseg)])

    agg = pl.kernel(
        _agg,
        out_type=jax.ShapeDtypeStruct((NC, apad, d), jnp.float32),
        mesh=mesh,
        scratch_types=[
            [pltpu.VMEM((2, CHUNK), jnp.int32) for _ in range(NIB)],
            [pltpu.VMEM((CHUNK, d), jnp.float32) for _ in range(NBUF)],
            pltpu.VMEM_SHARED((apad, d), jnp.float32),
            [pltpu.SemaphoreType.DMA for _ in range(NIB)],
            [pltpu.SemaphoreType.DMA for _ in range(NBUF)],
            [pltpu.SemaphoreType.DMA for _ in range(NBUF)],
        ],
    )
    accp = agg(ys, e4)

    def _fin_body(accp_ref, inv_ref, b_ref, out_ref):
        a = accp_ref[0] + accp_ref[1]
        out_ref[...] = a[:out_ref.shape[0]] * inv_ref[...] + b_ref[...]

    out = pl.pallas_call(
        _fin_body,
        out_shape=jax.ShapeDtypeStruct((n, d), jnp.float32),
    )(accp, inv_col, b.reshape(1, d))
    return out
